# Initial kernel scaffold; baseline (speedup 1.0000x reference)
import jax
import jax.numpy as jnp
from jax.experimental import pallas as pl


def kernel(probs, gt, img_size):
    raise NotImplementedError("write your pallas kernel here")


# speedup vs baseline: 14.8218x; 14.8218x over previous
import functools

import jax
import jax.numpy as jnp
import numpy as np
from jax import lax
from jax.experimental import pallas as pl
from jax.experimental.pallas import tpu as pltpu

_H = 2048
_W = 128
_N = _H * _W
_ROWS = 16

_LOG_EPS = np.float32(np.log(1e-7))
_LOG_1MEPS = np.float32(np.log(1.0 - 1e-7))
_I0 = np.int32(0)


def _row_kernel(k_ref, p0_ref, p1_ref, gt_ref, out_ref, loss_ref):
    d = p1_ref[0] - p0_ref[0]
    sp_tail = jnp.log1p(jnp.exp(-jnp.abs(d)))
    sp_d = jnp.maximum(d, 0.0) + sp_tail
    sp_nd = sp_d - d
    log_pc = jnp.clip(-sp_nd, _LOG_EPS, _LOG_1MEPS)
    log_1mpc = jnp.clip(-sp_d, _LOG_EPS, _LOG_1MEPS)
    g = gt_ref[0]
    loss_ref[...] = -(g * log_pc) - (1.0 - g) * log_1mpc

    k = k_ref[0]
    kf = k.astype(jnp.float32)
    bits = lax.bitcast_convert_type(loss_ref[...], jnp.int32)
    one = jnp.float32(1.0)
    zero = jnp.float32(0.0)

    def body(_, carry):
        lo, hi = carry
        mid = lo + ((hi - lo) >> 1)
        cnt = jnp.sum(jnp.where(bits >= mid, one, zero))
        pred = cnt >= kf
        return jnp.where(pred, mid, lo), jnp.where(pred, hi, mid)

    lo, _ = lax.fori_loop(
        np.int32(0), np.int32(31), body, (jnp.int32(0), jnp.int32(0x7F7FFFFF))
    )
    t_bits = lo
    t_val = lax.bitcast_convert_type(t_bits, jnp.float32)
    gt_mask = bits > t_bits
    cnt_gt = jnp.sum(jnp.where(gt_mask, one, zero))
    sum_gt = jnp.sum(jnp.where(gt_mask, loss_ref[...], zero))
    row_sum = sum_gt + (kf - cnt_gt) * t_val
    row_sum = jnp.where(k <= 0, jnp.float32(0.0), row_sum)
    out_ref[...] = jnp.full((1, 1, _W), row_sum, jnp.float32)


@jax.jit
def _topk_mean(probs, gt, k_arr):
    p0 = probs[:, 0].reshape(_ROWS, _H, _W)
    p1 = probs[:, 1].reshape(_ROWS, _H, _W)
    gtr = gt.reshape(_ROWS, _H, _W)

    grid_spec = pltpu.PrefetchScalarGridSpec(
        num_scalar_prefetch=1,
        grid=(_ROWS,),
        in_specs=[
            pl.BlockSpec((1, _H, _W), lambda i, k: (i, _I0, _I0)),
            pl.BlockSpec((1, _H, _W), lambda i, k: (i, _I0, _I0)),
            pl.BlockSpec((1, _H, _W), lambda i, k: (i, _I0, _I0)),
        ],
        out_specs=pl.BlockSpec((1, 1, _W), lambda i, k: (i, _I0, _I0)),
        scratch_shapes=[pltpu.VMEM((_H, _W), jnp.float32)],
    )
    row_sums = pl.pallas_call(
        _row_kernel,
        grid_spec=grid_spec,
        out_shape=jax.ShapeDtypeStruct((_ROWS, 1, _W), jnp.float32),
        compiler_params=pltpu.CompilerParams(
            dimension_semantics=("arbitrary",),
        ),
    )(k_arr, p0, p1, gtr)
    return jnp.sum(row_sums[:, 0, 0]) / np.float32(_ROWS * _N)


def kernel(probs, gt, img_size):
    k = (img_size[0].astype(jnp.int32) * img_size[1].astype(jnp.int32) * 90) // 100
    k_arr = k.reshape(1).astype(jnp.int32)
    return _topk_mean(probs, gt, k_arr)


# speedup vs baseline: 15.8327x; 1.0682x over previous
import functools

import jax
import jax.numpy as jnp
import numpy as np
from jax import lax
from jax.experimental import pallas as pl
from jax.experimental.pallas import tpu as pltpu
from jax.experimental.pallas import tpu_sc as plsc

_H = 2048
_W = 128
_N = _H * _W
_ROWS = 16
_HALF = _N // 2
_CH = 16384
_NCHUNK = _HALF // _CH
_NB = 2048

_LOG_EPS = np.float32(np.log(1e-7))
_LOG_1MEPS = np.float32(np.log(1.0 - 1e-7))
_I0 = np.int32(0)


def _loop32(n, body, init):
    def sb(carry, _):
        j, st = carry
        return (j + np.int32(1), body(j, st)), None

    (_, out), _ = lax.scan(sb, (jnp.int32(0), init), None, length=n)
    return out


def _loss_kernel(p0_ref, p1_ref, gt_ref, out_ref):
    d = p1_ref[0] - p0_ref[0]
    sp_tail = jnp.log1p(jnp.exp(-jnp.abs(d)))
    sp_d = jnp.maximum(d, 0.0) + sp_tail
    sp_nd = sp_d - d
    log_pc = jnp.clip(-sp_nd, _LOG_EPS, _LOG_1MEPS)
    log_1mpc = jnp.clip(-sp_d, _LOG_EPS, _LOG_1MEPS)
    g = gt_ref[0]
    out_ref[0] = -(g * log_pc) - (1.0 - g) * log_1mpc


def _select_body(loss_hbm, kf_hbm, out_hbm, buf, hist_cnt, hist_sum, red,
                 comb, kf_v, outv, shared):
    s = lax.axis_index("s")
    c = lax.axis_index("c")
    i32 = np.int32
    half = s % i32(2)
    lr = s // i32(2)
    r = c * i32(8) + lr
    base = r * i32(_N) + half * i32(_HALF)

    iota = lax.iota(jnp.int32, 16)
    zeros16 = jnp.zeros((16,), jnp.float32)
    ones16 = jnp.ones((16,), jnp.float32)
    izeros16 = jnp.zeros((16,), jnp.int32)

    pltpu.sync_copy(kf_hbm, kf_v)
    kf = kf_v[...]

    def zero_hists():
        def zb(j, carry):
            hist_cnt[pl.ds(j * i32(16), 16)] = zeros16
            hist_sum[pl.ds(j * i32(16), 16)] = zeros16
            return carry

        _loop32(2 * _NB * 16 // 32, zb, jnp.int32(0))

    def scatter_pass(bstar_vec):
        for chunk in range(_NCHUNK):
            pltpu.sync_copy(loss_hbm.at[pl.ds(base + i32(chunk * _CH), _CH)], buf)

            def body(j, carry):
                v = buf[pl.ds(j * i32(16), 16)]
                bits = plsc.bitcast(v, jnp.int32)
                b_top = bits >> i32(20)
                if bstar_vec is None:
                    idx = iota * i32(_NB) + b_top
                    plsc.addupdate_scatter(hist_cnt, [idx], ones16)
                    plsc.addupdate_scatter(hist_sum, [idx], v)
                else:
                    m = b_top == bstar_vec
                    sb = (bits >> i32(9)) & i32(_NB - 1)
                    idx = iota * i32(_NB) + sb
                    plsc.addupdate_scatter(hist_cnt, [idx], ones16, mask=m)
                    plsc.addupdate_scatter(hist_sum, [idx], v, mask=m)
                return carry

            _loop32(_CH // 16, body, jnp.int32(0))

    def lane_reduce_and_stage():
        def body(j, carry):
            acc_c = zeros16
            acc_s = zeros16
            for l in range(16):
                off = i32(l * _NB) + j * i32(16)
                acc_c = acc_c + hist_cnt[pl.ds(off, 16)]
                acc_s = acc_s + hist_sum[pl.ds(off, 16)]
            red[0, pl.ds(j * i32(16), 16)] = acc_c
            red[1, pl.ds(j * i32(16), 16)] = acc_s
            return carry

        _loop32(_NB // 16, body, jnp.int32(0))
        pltpu.sync_copy(red, shared.at[s])

    def scan_desc(kf_vec):
        def body(i, st):
            found, bbase, selc, sels, selS, selSS, selpc, carc, cars = st
            j = i32(_NB // 16 - 1) - i
            c_cnt = (comb[0, 0, pl.ds(j * i32(16), 16)]
                     + comb[1, 0, pl.ds(j * i32(16), 16)])
            c_sum = (comb[0, 1, pl.ds(j * i32(16), 16)]
                     + comb[1, 1, pl.ds(j * i32(16), 16)])
            sfx_c = jnp.flip(plsc.cumsum(jnp.flip(c_cnt))) + carc
            sfx_s = jnp.flip(plsc.cumsum(jnp.flip(c_sum))) + cars
            mask = sfx_c >= kf_vec
            pc = plsc.all_reduce_population_count(mask)
            hit = (pc > i32(0)) & (found == i32(0))
            found = jnp.where(hit, jnp.int32(1), found)
            bbase = jnp.where(hit, jnp.full((16,), j, jnp.int32), bbase)
            selc = jnp.where(hit, c_cnt, selc)
            sels = jnp.where(hit, c_sum, sels)
            selS = jnp.where(hit, sfx_c, selS)
            selSS = jnp.where(hit, sfx_s, selSS)
            selpc = jnp.where(hit, pc, selpc)
            carc = carc + jnp.full((16,), jnp.sum(c_cnt), jnp.float32)
            cars = cars + jnp.full((16,), jnp.sum(c_sum), jnp.float32)
            return (found, bbase, selc, sels, selS, selSS, selpc, carc, cars)

        st0 = (izeros16, izeros16, zeros16, zeros16, zeros16, zeros16,
               izeros16, zeros16, zeros16)
        (_, bbase, selc, sels, selS, selSS, selpc, _, _) = _loop32(
            _NB // 16, body, st0)
        lane = selpc - i32(1)
        lm = iota == lane
        cntb = jnp.full((16,), jnp.sum(jnp.where(lm, selc, zeros16)),
                        jnp.float32)
        sumb = jnp.full((16,), jnp.sum(jnp.where(lm, sels, zeros16)),
                        jnp.float32)
        s_at = jnp.full((16,), jnp.sum(jnp.where(lm, selS, zeros16)),
                        jnp.float32)
        ss_at = jnp.full((16,), jnp.sum(jnp.where(lm, selSS, zeros16)),
                         jnp.float32)
        bstar = bbase * i32(16) + lane
        return bstar, s_at - cntb, ss_at - sumb, cntb, sumb

    pair = s - half

    zero_hists()
    scatter_pass(None)
    lane_reduce_and_stage()
    plsc.subcore_barrier()

    pltpu.sync_copy(shared.at[pl.ds(pair, 2)], comb)
    bstar, cnt_gt, sum_gt, _, _ = scan_desc(kf)
    plsc.subcore_barrier()

    zero_hists()
    scatter_pass(bstar)
    lane_reduce_and_stage()
    plsc.subcore_barrier()

    pltpu.sync_copy(shared.at[pl.ds(pair, 2)], comb)
    kf2 = kf - cnt_gt
    sb_star, cnt2_gt, sum2_gt, cnt2b, sum2b = scan_desc(kf2)
    remainder = kf2 - cnt2_gt
    avg = sum2b / jnp.maximum(cnt2b, 1.0)
    row = sum_gt + sum2_gt + remainder * avg
    row = jnp.where(kf <= 0.0, zeros16, row)

    @pl.when(half == 0)
    def _():
        outv[...] = row
        pltpu.sync_copy(outv, out_hbm.at[pl.ds(r * i32(16), 16)])


@jax.jit
def _run_pipeline(probs, gt, kf_arr):
    p0 = probs[:, 0].reshape(_ROWS, _H, _W)
    p1 = probs[:, 1].reshape(_ROWS, _H, _W)
    gtr = gt.reshape(_ROWS, _H, _W)

    loss = pl.pallas_call(
        _loss_kernel,
        grid=(_ROWS,),
        in_specs=[
            pl.BlockSpec((1, _H, _W), lambda i: (i, _I0, _I0)),
            pl.BlockSpec((1, _H, _W), lambda i: (i, _I0, _I0)),
            pl.BlockSpec((1, _H, _W), lambda i: (i, _I0, _I0)),
        ],
        out_specs=pl.BlockSpec((1, _H, _W), lambda i: (i, _I0, _I0)),
        out_shape=jax.ShapeDtypeStruct((_ROWS, _H, _W), jnp.float32),
        compiler_params=pltpu.CompilerParams(
            dimension_semantics=("arbitrary",),
        ),
    )(p0, p1, gtr)
    loss_flat = loss.reshape(_ROWS * _N)

    mesh = plsc.VectorSubcoreMesh(core_axis_name="c", subcore_axis_name="s")
    select = pl.kernel(
        _select_body,
        mesh=mesh,
        compiler_params=pltpu.CompilerParams(needs_layout_passes=False),
        out_type=jax.ShapeDtypeStruct((_ROWS * 16,), jnp.float32),
        scratch_types=[
            pltpu.VMEM((_CH,), jnp.float32),
            pltpu.VMEM((16 * _NB,), jnp.float32),
            pltpu.VMEM((16 * _NB,), jnp.float32),
            pltpu.VMEM((2, _NB), jnp.float32),
            pltpu.VMEM((2, 2, _NB), jnp.float32),
            pltpu.VMEM((16,), jnp.float32),
            pltpu.VMEM((16,), jnp.float32),
            pltpu.VMEM_SHARED((16, 2, _NB), jnp.float32),
        ],
    )
    return select(loss_flat, kf_arr)


@jax.jit
def _topk_mean(probs, gt, kf_arr):
    row_sums = _run_pipeline(probs, gt, kf_arr)
    return jnp.sum(row_sums.reshape(_ROWS, 16)[:, 0]) / np.float32(_ROWS * _N)


def kernel(probs, gt, img_size):
    k = (img_size[0].astype(jnp.int32) * img_size[1].astype(jnp.int32) * 90) // 100
    kf_arr = jnp.full((16,), k.astype(jnp.float32), dtype=jnp.float32)
    return _topk_mean(probs, gt, kf_arr)


# speedup vs baseline: 18.3700x; 1.1603x over previous
import functools

import jax
import jax.numpy as jnp
import numpy as np
from jax import lax
from jax.experimental import pallas as pl
from jax.experimental.pallas import tpu as pltpu
from jax.experimental.pallas import tpu_sc as plsc

_H = 2048
_W = 128
_N = _H * _W
_ROWS = 16
_HALF = _N // 2
_CH = 16384
_NCHUNK = _HALF // _CH
_NB = 2048

_LOG_EPS = np.float32(np.log(1e-7))
_LOG_1MEPS = np.float32(np.log(1.0 - 1e-7))
_I0 = np.int32(0)


def _loop32(n, body, init):
    def sb(carry, _):
        j, st = carry
        return (j + np.int32(1), body(j, st)), None

    (_, out), _ = lax.scan(sb, (jnp.int32(0), init), None, length=n)
    return out


def _loss_kernel(p0_ref, p1_ref, gt_ref, out_ref):
    d = p1_ref[0] - p0_ref[0]
    sp_tail = jnp.log1p(jnp.exp(-jnp.abs(d)))
    sp_d = jnp.maximum(d, 0.0) + sp_tail
    sp_nd = sp_d - d
    log_pc = jnp.clip(-sp_nd, _LOG_EPS, _LOG_1MEPS)
    log_1mpc = jnp.clip(-sp_d, _LOG_EPS, _LOG_1MEPS)
    g = gt_ref[0]
    out_ref[0] = -(g * log_pc) - (1.0 - g) * log_1mpc


def _select_body(loss_hbm, kf_hbm, out_hbm, buf0, buf1, hist_cnt, hist_sum,
                 red, comb, kf_v, outv, shared, sem0, sem1):
    s = lax.axis_index("s")
    c = lax.axis_index("c")
    i32 = np.int32
    half = s % i32(2)
    lr = s // i32(2)
    r = c * i32(8) + lr
    base = r * i32(_N) + half * i32(_HALF)

    iota = lax.iota(jnp.int32, 16)
    zeros16 = jnp.zeros((16,), jnp.float32)
    ones16 = jnp.ones((16,), jnp.float32)
    izeros16 = jnp.zeros((16,), jnp.int32)

    pltpu.sync_copy(kf_hbm, kf_v)
    kf = kf_v[...]

    def zero_hists(both):
        def zb(j, carry):
            for u in range(4):
                off = j * i32(64) + i32(u * 16)
                hist_cnt[pl.ds(off, 16)] = zeros16
                if both:
                    hist_sum[pl.ds(off, 16)] = zeros16
            return carry

        _loop32(_NB * 16 // 64, zb, jnp.int32(0))

    def scatter_pass(bstar_vec):
        bufs = (buf0, buf1)
        sems = (sem0, sem1)
        handles = {}
        handles[0] = pltpu.async_copy(
            loss_hbm.at[pl.ds(base, _CH)], bufs[0], sems[0])
        acc = (zeros16, zeros16)
        for chunk in range(_NCHUNK):
            cur = chunk % 2
            if chunk + 1 < _NCHUNK:
                handles[chunk + 1] = pltpu.async_copy(
                    loss_hbm.at[pl.ds(base + i32((chunk + 1) * _CH), _CH)],
                    bufs[(chunk + 1) % 2], sems[(chunk + 1) % 2])
            handles[chunk].wait()
            bref = bufs[cur]

            def body(j, carry):
                ac, asum = carry
                for u in range(4):
                    off = j * i32(64) + i32(u * 16)
                    v = bref[pl.ds(off, 16)]
                    bits = plsc.bitcast(v, jnp.int32)
                    b_top = bits >> i32(20)
                    if bstar_vec is None:
                        idx = iota * i32(_NB) + b_top
                        plsc.addupdate_scatter(hist_cnt, [idx], ones16)
                    else:
                        m = b_top == bstar_vec
                        gm = b_top > bstar_vec
                        sb = (bits >> i32(9)) & i32(_NB - 1)
                        idx = iota * i32(_NB) + sb
                        plsc.addupdate_scatter(hist_cnt, [idx], ones16,
                                               mask=m)
                        plsc.addupdate_scatter(hist_sum, [idx], v, mask=m)
                        ac = ac + jnp.where(gm, ones16, zeros16)
                        asum = asum + jnp.where(gm, v, zeros16)
                return (ac, asum)

            acc = _loop32(_CH // 64, body, acc)
        return acc

    def lane_reduce_and_stage(acc, do_sum):
        def body(j, carry):
            acc_c = zeros16
            acc_s = zeros16
            for l in range(16):
                off = i32(l * _NB) + j * i32(16)
                acc_c = acc_c + hist_cnt[pl.ds(off, 16)]
                if do_sum:
                    acc_s = acc_s + hist_sum[pl.ds(off, 16)]
            red[0, pl.ds(j * i32(16), 16)] = acc_c
            if do_sum:
                red[1, pl.ds(j * i32(16), 16)] = acc_s
            return carry

        _loop32(_NB // 16, body, jnp.int32(0))
        red[0, pl.ds(_NB, 16)] = acc[0]
        red[1, pl.ds(_NB, 16)] = acc[1]
        pltpu.sync_copy(red, shared.at[s])

    def scan_desc(kf_vec, with_sums):
        def body(i, st):
            found, bbase, selc, sels, selS, selSS, selpc, carc, cars = st
            j = i32(_NB // 16 - 1) - i
            c_cnt = (comb[0, 0, pl.ds(j * i32(16), 16)]
                     + comb[1, 0, pl.ds(j * i32(16), 16)])
            sfx_c = jnp.flip(plsc.cumsum(jnp.flip(c_cnt))) + carc
            mask = sfx_c >= kf_vec
            pc = plsc.all_reduce_population_count(mask)
            hit = (pc > i32(0)) & (found == i32(0))
            found = jnp.where(hit, jnp.int32(1), found)
            bbase = jnp.where(hit, jnp.full((16,), j, jnp.int32), bbase)
            selc = jnp.where(hit, c_cnt, selc)
            selS = jnp.where(hit, sfx_c, selS)
            selpc = jnp.where(hit, pc, selpc)
            carc = carc + jnp.full((16,), jnp.sum(c_cnt), jnp.float32)
            if with_sums:
                c_sum = (comb[0, 1, pl.ds(j * i32(16), 16)]
                         + comb[1, 1, pl.ds(j * i32(16), 16)])
                sfx_s = jnp.flip(plsc.cumsum(jnp.flip(c_sum))) + cars
                sels = jnp.where(hit, c_sum, sels)
                selSS = jnp.where(hit, sfx_s, selSS)
                cars = cars + jnp.full((16,), jnp.sum(c_sum), jnp.float32)
            return (found, bbase, selc, sels, selS, selSS, selpc, carc, cars)

        st0 = (izeros16, izeros16, zeros16, zeros16, zeros16, zeros16,
               izeros16, zeros16, zeros16)
        (_, bbase, selc, sels, selS, selSS, selpc, _, _) = _loop32(
            _NB // 16, body, st0)
        lane = selpc - i32(1)
        lm = iota == lane
        cntb = jnp.full((16,), jnp.sum(jnp.where(lm, selc, zeros16)),
                        jnp.float32)
        sumb = jnp.full((16,), jnp.sum(jnp.where(lm, sels, zeros16)),
                        jnp.float32)
        s_at = jnp.full((16,), jnp.sum(jnp.where(lm, selS, zeros16)),
                        jnp.float32)
        ss_at = jnp.full((16,), jnp.sum(jnp.where(lm, selSS, zeros16)),
                         jnp.float32)
        bstar = bbase * i32(16) + lane
        return bstar, s_at - cntb, ss_at - sumb, cntb, sumb

    pair = s - half

    zero_hists(both=False)
    scatter_pass(None)
    lane_reduce_and_stage((zeros16, zeros16), do_sum=False)
    plsc.subcore_barrier()

    pltpu.sync_copy(shared.at[pl.ds(pair, 2)], comb)
    bstar, _, _, _, _ = scan_desc(kf, with_sums=False)
    plsc.subcore_barrier()

    zero_hists(both=True)
    acc = scatter_pass(bstar)
    lane_reduce_and_stage(acc, do_sum=True)
    plsc.subcore_barrier()

    pltpu.sync_copy(shared.at[pl.ds(pair, 2)], comb)
    tail_c = comb[0, 0, pl.ds(_NB, 16)] + comb[1, 0, pl.ds(_NB, 16)]
    tail_s = comb[0, 1, pl.ds(_NB, 16)] + comb[1, 1, pl.ds(_NB, 16)]
    cnt_gt = jnp.full((16,), jnp.sum(tail_c), jnp.float32)
    sum_gt = jnp.full((16,), jnp.sum(tail_s), jnp.float32)
    kf2 = kf - cnt_gt
    sb_star, cnt2_gt, sum2_gt, cnt2b, sum2b = scan_desc(kf2, with_sums=True)
    remainder = kf2 - cnt2_gt
    avg = sum2b / jnp.maximum(cnt2b, 1.0)
    row = sum_gt + sum2_gt + remainder * avg
    row = jnp.where(kf <= 0.0, zeros16, row)

    @pl.when(half == 0)
    def _():
        outv[...] = row
        pltpu.sync_copy(outv, out_hbm.at[pl.ds(r * i32(16), 16)])


@jax.jit
def _run_pipeline(probs, gt, kf_arr):
    p0 = probs[:, 0].reshape(_ROWS, _H, _W)
    p1 = probs[:, 1].reshape(_ROWS, _H, _W)
    gtr = gt.reshape(_ROWS, _H, _W)

    loss = pl.pallas_call(
        _loss_kernel,
        grid=(_ROWS,),
        in_specs=[
            pl.BlockSpec((1, _H, _W), lambda i: (i, _I0, _I0)),
            pl.BlockSpec((1, _H, _W), lambda i: (i, _I0, _I0)),
            pl.BlockSpec((1, _H, _W), lambda i: (i, _I0, _I0)),
        ],
        out_specs=pl.BlockSpec((1, _H, _W), lambda i: (i, _I0, _I0)),
        out_shape=jax.ShapeDtypeStruct((_ROWS, _H, _W), jnp.float32),
        compiler_params=pltpu.CompilerParams(
            dimension_semantics=("arbitrary",),
        ),
    )(p0, p1, gtr)
    loss_flat = loss.reshape(_ROWS * _N)

    mesh = plsc.VectorSubcoreMesh(core_axis_name="c", subcore_axis_name="s")
    select = pl.kernel(
        _select_body,
        mesh=mesh,
        compiler_params=pltpu.CompilerParams(needs_layout_passes=False),
        out_type=jax.ShapeDtypeStruct((_ROWS * 16,), jnp.float32),
        scratch_types=[
            pltpu.VMEM((_CH,), jnp.float32),
            pltpu.VMEM((_CH,), jnp.float32),
            pltpu.VMEM((16 * _NB,), jnp.float32),
            pltpu.VMEM((16 * _NB,), jnp.float32),
            pltpu.VMEM((2, _NB + 16), jnp.float32),
            pltpu.VMEM((2, 2, _NB + 16), jnp.float32),
            pltpu.VMEM((16,), jnp.float32),
            pltpu.VMEM((16,), jnp.float32),
            pltpu.VMEM_SHARED((16, 2, _NB + 16), jnp.float32),
            pltpu.SemaphoreType.DMA,
            pltpu.SemaphoreType.DMA,
        ],
    )
    return select(loss_flat, kf_arr)


@jax.jit
def _topk_mean(probs, gt, kf_arr):
    row_sums = _run_pipeline(probs, gt, kf_arr)
    return jnp.sum(row_sums.reshape(_ROWS, 16)[:, 0]) / np.float32(_ROWS * _N)


def kernel(probs, gt, img_size):
    k = (img_size[0].astype(jnp.int32) * img_size[1].astype(jnp.int32) * 90) // 100
    kf_arr = jnp.full((16,), k.astype(jnp.float32), dtype=jnp.float32)
    return _topk_mean(probs, gt, kf_arr)


# speedup vs baseline: 18.5397x; 1.0092x over previous
import functools

import jax
import jax.numpy as jnp
import numpy as np
from jax import lax
from jax.experimental import pallas as pl
from jax.experimental.pallas import tpu as pltpu
from jax.experimental.pallas import tpu_sc as plsc

_H = 2048
_W = 128
_N = _H * _W
_ROWS = 16
_HALF = _N // 2
_CH = 16384
_NCHUNK = _HALF // _CH
_NB = 2048
_LS = _NB + 1
_HW = 513 * 64

_LOG_EPS = np.float32(np.log(1e-7))
_LOG_1MEPS = np.float32(np.log(1.0 - 1e-7))
_I0 = np.int32(0)


def _loop32(n, body, init):
    def sb(carry, _):
        j, st = carry
        return (j + np.int32(1), body(j, st)), None

    (_, out), _ = lax.scan(sb, (jnp.int32(0), init), None, length=n)
    return out


def _loss_kernel(p0_ref, p1_ref, gt_ref, out_ref):
    d = p1_ref[0] - p0_ref[0]
    sp_tail = jnp.log1p(jnp.exp(-jnp.abs(d)))
    sp_d = jnp.maximum(d, 0.0) + sp_tail
    sp_nd = sp_d - d
    log_pc = jnp.clip(-sp_nd, _LOG_EPS, _LOG_1MEPS)
    log_1mpc = jnp.clip(-sp_d, _LOG_EPS, _LOG_1MEPS)
    g = gt_ref[0]
    out_ref[0] = -(g * log_pc) - (1.0 - g) * log_1mpc


def _select_body(loss_hbm, kf_hbm, out_hbm, buf0, buf1, hist_cnt, hist_sum,
                 red, comb, kf_v, outv, shared, sem0, sem1):
    s = lax.axis_index("s")
    c = lax.axis_index("c")
    i32 = np.int32
    half = s % i32(2)
    lr = s // i32(2)
    r = c * i32(8) + lr
    base = r * i32(_N) + half * i32(_HALF)

    iota = lax.iota(jnp.int32, 16)
    zeros16 = jnp.zeros((16,), jnp.float32)
    ones16 = jnp.ones((16,), jnp.float32)
    izeros16 = jnp.zeros((16,), jnp.int32)

    pltpu.sync_copy(kf_hbm, kf_v)
    kf = kf_v[...]

    def zero_hists(both):
        def zb(j, carry):
            for u in range(4):
                off = j * i32(64) + i32(u * 16)
                hist_cnt[pl.ds(off, 16)] = zeros16
                if both:
                    hist_sum[pl.ds(off, 16)] = zeros16
            return carry

        _loop32(_HW // 64, zb, jnp.int32(0))

    def scatter_pass(bstar_vec):
        bufs = (buf0, buf1)
        sems = (sem0, sem1)
        handles = {}
        handles[0] = pltpu.async_copy(
            loss_hbm.at[pl.ds(base, _CH)], bufs[0], sems[0])
        acc = (zeros16, zeros16)
        for chunk in range(_NCHUNK):
            cur = chunk % 2
            if chunk + 1 < _NCHUNK:
                handles[chunk + 1] = pltpu.async_copy(
                    loss_hbm.at[pl.ds(base + i32((chunk + 1) * _CH), _CH)],
                    bufs[(chunk + 1) % 2], sems[(chunk + 1) % 2])
            handles[chunk].wait()
            bref = bufs[cur]

            def body(j, carry):
                ac, asum = carry
                for u in range(4):
                    off = j * i32(64) + i32(u * 16)
                    v = bref[pl.ds(off, 16)]
                    bits = plsc.bitcast(v, jnp.int32)
                    b_top = bits >> i32(20)
                    if bstar_vec is None:
                        idx = iota * i32(_LS) + b_top
                        plsc.addupdate_scatter(hist_cnt, [idx], ones16)
                    else:
                        m = b_top == bstar_vec
                        gm = b_top > bstar_vec
                        sb = (bits >> i32(9)) & i32(_NB - 1)
                        idx = iota * i32(_LS) + sb
                        plsc.addupdate_scatter(hist_cnt, [idx], ones16,
                                               mask=m)
                        plsc.addupdate_scatter(hist_sum, [idx], v, mask=m)
                        ac = ac + jnp.where(gm, ones16, zeros16)
                        asum = asum + jnp.where(gm, v, zeros16)
                return (ac, asum)

            acc = _loop32(_CH // 64, body, acc)
        return acc

    def lane_reduce_and_stage(acc, do_sum):
        def body(j, carry):
            acc_c = zeros16
            acc_s = zeros16
            for l in range(16):
                off = i32(l * _LS) + j * i32(16)
                acc_c = acc_c + hist_cnt[pl.ds(off, 16)]
                if do_sum:
                    acc_s = acc_s + hist_sum[pl.ds(off, 16)]
            red[0, pl.ds(j * i32(16), 16)] = acc_c
            if do_sum:
                red[1, pl.ds(j * i32(16), 16)] = acc_s
            return carry

        _loop32(_NB // 16, body, jnp.int32(0))
        red[0, pl.ds(_NB, 16)] = acc[0]
        red[1, pl.ds(_NB, 16)] = acc[1]
        pltpu.sync_copy(red, shared.at[s])

    def scan_desc(kf_vec, with_sums):
        def body(i, st):
            found, bbase, selc, sels, selS, selSS, selpc, carc, cars = st
            j = i32(_NB // 16 - 1) - i
            c_cnt = (comb[0, 0, pl.ds(j * i32(16), 16)]
                     + comb[1, 0, pl.ds(j * i32(16), 16)])
            sfx_c = jnp.flip(plsc.cumsum(jnp.flip(c_cnt))) + carc
            mask = sfx_c >= kf_vec
            pc = plsc.all_reduce_population_count(mask)
            hit = (pc > i32(0)) & (found == i32(0))
            found = jnp.where(hit, jnp.int32(1), found)
            bbase = jnp.where(hit, jnp.full((16,), j, jnp.int32), bbase)
            selc = jnp.where(hit, c_cnt, selc)
            selS = jnp.where(hit, sfx_c, selS)
            selpc = jnp.where(hit, pc, selpc)
            carc = carc + jnp.full((16,), jnp.sum(c_cnt), jnp.float32)
            if with_sums:
                c_sum = (comb[0, 1, pl.ds(j * i32(16), 16)]
                         + comb[1, 1, pl.ds(j * i32(16), 16)])
                sfx_s = jnp.flip(plsc.cumsum(jnp.flip(c_sum))) + cars
                sels = jnp.where(hit, c_sum, sels)
                selSS = jnp.where(hit, sfx_s, selSS)
                cars = cars + jnp.full((16,), jnp.sum(c_sum), jnp.float32)
            return (found, bbase, selc, sels, selS, selSS, selpc, carc, cars)

        st0 = (izeros16, izeros16, zeros16, zeros16, zeros16, zeros16,
               izeros16, zeros16, zeros16)
        (_, bbase, selc, sels, selS, selSS, selpc, _, _) = _loop32(
            _NB // 16, body, st0)
        lane = selpc - i32(1)
        lm = iota == lane
        cntb = jnp.full((16,), jnp.sum(jnp.where(lm, selc, zeros16)),
                        jnp.float32)
        sumb = jnp.full((16,), jnp.sum(jnp.where(lm, sels, zeros16)),
                        jnp.float32)
        s_at = jnp.full((16,), jnp.sum(jnp.where(lm, selS, zeros16)),
                        jnp.float32)
        ss_at = jnp.full((16,), jnp.sum(jnp.where(lm, selSS, zeros16)),
                         jnp.float32)
        bstar = bbase * i32(16) + lane
        return bstar, s_at - cntb, ss_at - sumb, cntb, sumb

    pair = s - half

    zero_hists(both=False)
    scatter_pass(None)
    lane_reduce_and_stage((zeros16, zeros16), do_sum=False)
    plsc.subcore_barrier()

    pltpu.sync_copy(shared.at[pl.ds(pair, 2)], comb)
    bstar, _, _, _, _ = scan_desc(kf, with_sums=False)
    plsc.subcore_barrier()

    zero_hists(both=True)
    acc = scatter_pass(bstar)
    lane_reduce_and_stage(acc, do_sum=True)
    plsc.subcore_barrier()

    pltpu.sync_copy(shared.at[pl.ds(pair, 2)], comb)
    tail_c = comb[0, 0, pl.ds(_NB, 16)] + comb[1, 0, pl.ds(_NB, 16)]
    tail_s = comb[0, 1, pl.ds(_NB, 16)] + comb[1, 1, pl.ds(_NB, 16)]
    cnt_gt = jnp.full((16,), jnp.sum(tail_c), jnp.float32)
    sum_gt = jnp.full((16,), jnp.sum(tail_s), jnp.float32)
    kf2 = kf - cnt_gt
    sb_star, cnt2_gt, sum2_gt, cnt2b, sum2b = scan_desc(kf2, with_sums=True)
    remainder = kf2 - cnt2_gt
    avg = sum2b / jnp.maximum(cnt2b, 1.0)
    row = sum_gt + sum2_gt + remainder * avg
    row = jnp.where(kf <= 0.0, zeros16, row)

    @pl.when(half == 0)
    def _():
        outv[...] = row
        pltpu.sync_copy(outv, out_hbm.at[pl.ds(r * i32(16), 16)])


@jax.jit
def _run_pipeline(probs, gt, kf_arr):
    p0 = probs[:, 0].reshape(_ROWS, _H, _W)
    p1 = probs[:, 1].reshape(_ROWS, _H, _W)
    gtr = gt.reshape(_ROWS, _H, _W)

    loss = pl.pallas_call(
        _loss_kernel,
        grid=(_ROWS,),
        in_specs=[
            pl.BlockSpec((1, _H, _W), lambda i: (i, _I0, _I0)),
            pl.BlockSpec((1, _H, _W), lambda i: (i, _I0, _I0)),
            pl.BlockSpec((1, _H, _W), lambda i: (i, _I0, _I0)),
        ],
        out_specs=pl.BlockSpec((1, _H, _W), lambda i: (i, _I0, _I0)),
        out_shape=jax.ShapeDtypeStruct((_ROWS, _H, _W), jnp.float32),
        compiler_params=pltpu.CompilerParams(
            dimension_semantics=("arbitrary",),
        ),
    )(p0, p1, gtr)
    loss_flat = loss.reshape(_ROWS * _N)

    mesh = plsc.VectorSubcoreMesh(core_axis_name="c", subcore_axis_name="s")
    select = pl.kernel(
        _select_body,
        mesh=mesh,
        compiler_params=pltpu.CompilerParams(needs_layout_passes=False),
        out_type=jax.ShapeDtypeStruct((_ROWS * 16,), jnp.float32),
        scratch_types=[
            pltpu.VMEM((_CH,), jnp.float32),
            pltpu.VMEM((_CH,), jnp.float32),
            pltpu.VMEM((_HW,), jnp.float32),
            pltpu.VMEM((_HW,), jnp.float32),
            pltpu.VMEM((2, _NB + 16), jnp.float32),
            pltpu.VMEM((2, 2, _NB + 16), jnp.float32),
            pltpu.VMEM((16,), jnp.float32),
            pltpu.VMEM((16,), jnp.float32),
            pltpu.VMEM_SHARED((16, 2, _NB + 16), jnp.float32),
            pltpu.SemaphoreType.DMA,
            pltpu.SemaphoreType.DMA,
        ],
    )
    return select(loss_flat, kf_arr)


@jax.jit
def _topk_mean(probs, gt, kf_arr):
    row_sums = _run_pipeline(probs, gt, kf_arr)
    return jnp.sum(row_sums.reshape(_ROWS, 16)[:, 0]) / np.float32(_ROWS * _N)


def kernel(probs, gt, img_size):
    k = (img_size[0].astype(jnp.int32) * img_size[1].astype(jnp.int32) * 90) // 100
    kf_arr = jnp.full((16,), k.astype(jnp.float32), dtype=jnp.float32)
    return _topk_mean(probs, gt, kf_arr)


# speedup vs baseline: 18.7390x; 1.0107x over previous
import functools

import jax
import jax.numpy as jnp
import numpy as np
from jax import lax
from jax.experimental import pallas as pl
from jax.experimental.pallas import tpu as pltpu
from jax.experimental.pallas import tpu_sc as plsc

_H = 2048
_W = 128
_N = _H * _W
_ROWS = 16
_HALF = _N // 2
_CH = 16384
_NCHUNK = _HALF // _CH
_NB = 2048
_LS = _NB + 1
_HW = 513 * 64

_LOG_EPS = np.float32(np.log(1e-7))
_LOG_1MEPS = np.float32(np.log(1.0 - 1e-7))
_I0 = np.int32(0)


def _loop32(n, body, init):
    def sb(carry, _):
        j, st = carry
        return (j + np.int32(1), body(j, st)), None

    (_, out), _ = lax.scan(sb, (jnp.int32(0), init), None, length=n)
    return out


def _loss_kernel(p0_ref, p1_ref, gt_ref, out_ref):
    d = p1_ref[0] - p0_ref[0]
    sp_tail = jnp.log1p(jnp.exp(-jnp.abs(d)))
    sp_d = jnp.maximum(d, 0.0) + sp_tail
    sp_nd = sp_d - d
    log_pc = jnp.clip(-sp_nd, _LOG_EPS, _LOG_1MEPS)
    log_1mpc = jnp.clip(-sp_d, _LOG_EPS, _LOG_1MEPS)
    g = gt_ref[0]
    out_ref[0] = -(g * log_pc) - (1.0 - g) * log_1mpc


def _select_body(loss_hbm, kf_hbm, out_hbm, buf0, buf1, hist_cnt, hist_sum,
                 red, comb, kf_v, outv, shared, sem0, sem1):
    s = lax.axis_index("s")
    c = lax.axis_index("c")
    i32 = np.int32
    half = s % i32(2)
    lr = s // i32(2)
    r = c * i32(8) + lr
    base = r * i32(_N) + half * i32(_HALF)

    iota = lax.iota(jnp.int32, 16)
    zeros16 = jnp.zeros((16,), jnp.float32)
    ones16 = jnp.ones((16,), jnp.float32)
    izeros16 = jnp.zeros((16,), jnp.int32)

    pltpu.sync_copy(kf_hbm, kf_v)
    kf = kf_v[...]

    def zero_hists(both):
        def zb(j, carry):
            for u in range(4):
                off = j * i32(64) + i32(u * 16)
                hist_cnt[pl.ds(off, 16)] = zeros16
                if both:
                    hist_sum[pl.ds(off, 16)] = zeros16
            return carry

        _loop32(_HW // 64, zb, jnp.int32(0))

    def scatter_pass(bstar_vec):
        bufs = (buf0, buf1)
        sems = (sem0, sem1)
        handles = {}
        handles[0] = pltpu.async_copy(
            loss_hbm.at[pl.ds(base, _CH)], bufs[0], sems[0])
        acc = (zeros16, zeros16)
        for chunk in range(_NCHUNK):
            cur = chunk % 2
            if chunk + 1 < _NCHUNK:
                handles[chunk + 1] = pltpu.async_copy(
                    loss_hbm.at[pl.ds(base + i32((chunk + 1) * _CH), _CH)],
                    bufs[(chunk + 1) % 2], sems[(chunk + 1) % 2])
            handles[chunk].wait()
            bref = bufs[cur]

            def body(j, carry):
                ac, asum = carry
                for u in range(8):
                    off = j * i32(128) + i32(u * 16)
                    v = bref[pl.ds(off, 16)]
                    bits = plsc.bitcast(v, jnp.int32)
                    b_top = bits >> i32(20)
                    if bstar_vec is None:
                        idx = iota * i32(_LS) + b_top
                        plsc.addupdate_scatter(hist_cnt, [idx], ones16)
                    else:
                        m = b_top == bstar_vec
                        gm = b_top > bstar_vec
                        sb = (bits >> i32(9)) & i32(_NB - 1)
                        idx = iota * i32(_LS) + sb
                        plsc.addupdate_scatter(hist_cnt, [idx], ones16,
                                               mask=m)
                        plsc.addupdate_scatter(hist_sum, [idx], v, mask=m)
                        ac = ac + jnp.where(gm, ones16, zeros16)
                        asum = asum + jnp.where(gm, v, zeros16)
                return (ac, asum)

            acc = _loop32(_CH // 128, body, acc)
        return acc

    def lane_reduce_and_stage(acc, do_sum):
        def body(j, carry):
            acc_c = zeros16
            acc_s = zeros16
            for l in range(16):
                off = i32(l * _LS) + j * i32(16)
                acc_c = acc_c + hist_cnt[pl.ds(off, 16)]
                if do_sum:
                    acc_s = acc_s + hist_sum[pl.ds(off, 16)]
                else:
                    hist_cnt[pl.ds(off, 16)] = zeros16
            red[0, pl.ds(j * i32(16), 16)] = acc_c
            if do_sum:
                red[1, pl.ds(j * i32(16), 16)] = acc_s
            return carry

        _loop32(_NB // 16, body, jnp.int32(0))
        red[0, pl.ds(_NB, 16)] = acc[0]
        red[1, pl.ds(_NB, 16)] = acc[1]
        pltpu.sync_copy(red, shared.at[s])

    def scan_desc(kf_vec, with_sums):
        def body(i, st):
            found, bbase, selc, sels, selS, selSS, selpc, carc, cars = st
            j = i32(_NB // 16 - 1) - i
            c_cnt = (comb[0, 0, pl.ds(j * i32(16), 16)]
                     + comb[1, 0, pl.ds(j * i32(16), 16)])
            sfx_c = jnp.flip(plsc.cumsum(jnp.flip(c_cnt))) + carc
            mask = sfx_c >= kf_vec
            pc = plsc.all_reduce_population_count(mask)
            hit = (pc > i32(0)) & (found == i32(0))
            found = jnp.where(hit, jnp.int32(1), found)
            bbase = jnp.where(hit, jnp.full((16,), j, jnp.int32), bbase)
            selc = jnp.where(hit, c_cnt, selc)
            selS = jnp.where(hit, sfx_c, selS)
            selpc = jnp.where(hit, pc, selpc)
            carc = carc + jnp.full((16,), jnp.sum(c_cnt), jnp.float32)
            if with_sums:
                c_sum = (comb[0, 1, pl.ds(j * i32(16), 16)]
                         + comb[1, 1, pl.ds(j * i32(16), 16)])
                sfx_s = jnp.flip(plsc.cumsum(jnp.flip(c_sum))) + cars
                sels = jnp.where(hit, c_sum, sels)
                selSS = jnp.where(hit, sfx_s, selSS)
                cars = cars + jnp.full((16,), jnp.sum(c_sum), jnp.float32)
            return (found, bbase, selc, sels, selS, selSS, selpc, carc, cars)

        st0 = (izeros16, izeros16, zeros16, zeros16, zeros16, zeros16,
               izeros16, zeros16, zeros16)
        (_, bbase, selc, sels, selS, selSS, selpc, _, _) = _loop32(
            _NB // 16, body, st0)
        lane = selpc - i32(1)
        lm = iota == lane
        cntb = jnp.full((16,), jnp.sum(jnp.where(lm, selc, zeros16)),
                        jnp.float32)
        sumb = jnp.full((16,), jnp.sum(jnp.where(lm, sels, zeros16)),
                        jnp.float32)
        s_at = jnp.full((16,), jnp.sum(jnp.where(lm, selS, zeros16)),
                        jnp.float32)
        ss_at = jnp.full((16,), jnp.sum(jnp.where(lm, selSS, zeros16)),
                         jnp.float32)
        bstar = bbase * i32(16) + lane
        return bstar, s_at - cntb, ss_at - sumb, cntb, sumb

    pair = s - half

    zero_hists(both=True)
    scatter_pass(None)
    lane_reduce_and_stage((zeros16, zeros16), do_sum=False)
    plsc.subcore_barrier()

    pltpu.sync_copy(shared.at[pl.ds(pair, 2)], comb)
    bstar, _, _, _, _ = scan_desc(kf, with_sums=False)
    plsc.subcore_barrier()

    acc = scatter_pass(bstar)
    lane_reduce_and_stage(acc, do_sum=True)
    plsc.subcore_barrier()

    pltpu.sync_copy(shared.at[pl.ds(pair, 2)], comb)
    tail_c = comb[0, 0, pl.ds(_NB, 16)] + comb[1, 0, pl.ds(_NB, 16)]
    tail_s = comb[0, 1, pl.ds(_NB, 16)] + comb[1, 1, pl.ds(_NB, 16)]
    cnt_gt = jnp.full((16,), jnp.sum(tail_c), jnp.float32)
    sum_gt = jnp.full((16,), jnp.sum(tail_s), jnp.float32)
    kf2 = kf - cnt_gt
    sb_star, cnt2_gt, sum2_gt, cnt2b, sum2b = scan_desc(kf2, with_sums=True)
    remainder = kf2 - cnt2_gt
    avg = sum2b / jnp.maximum(cnt2b, 1.0)
    row = sum_gt + sum2_gt + remainder * avg
    row = jnp.where(kf <= 0.0, zeros16, row)

    @pl.when(half == 0)
    def _():
        outv[...] = row
        pltpu.sync_copy(outv, out_hbm.at[pl.ds(r * i32(16), 16)])


@jax.jit
def _run_pipeline(probs, gt, kf_arr):
    p0 = probs[:, 0].reshape(_ROWS, _H, _W)
    p1 = probs[:, 1].reshape(_ROWS, _H, _W)
    gtr = gt.reshape(_ROWS, _H, _W)

    loss = pl.pallas_call(
        _loss_kernel,
        grid=(_ROWS,),
        in_specs=[
            pl.BlockSpec((1, _H, _W), lambda i: (i, _I0, _I0)),
            pl.BlockSpec((1, _H, _W), lambda i: (i, _I0, _I0)),
            pl.BlockSpec((1, _H, _W), lambda i: (i, _I0, _I0)),
        ],
        out_specs=pl.BlockSpec((1, _H, _W), lambda i: (i, _I0, _I0)),
        out_shape=jax.ShapeDtypeStruct((_ROWS, _H, _W), jnp.float32),
        compiler_params=pltpu.CompilerParams(
            dimension_semantics=("arbitrary",),
        ),
    )(p0, p1, gtr)
    loss_flat = loss.reshape(_ROWS * _N)

    mesh = plsc.VectorSubcoreMesh(core_axis_name="c", subcore_axis_name="s")
    select = pl.kernel(
        _select_body,
        mesh=mesh,
        compiler_params=pltpu.CompilerParams(needs_layout_passes=False),
        out_type=jax.ShapeDtypeStruct((_ROWS * 16,), jnp.float32),
        scratch_types=[
            pltpu.VMEM((_CH,), jnp.float32),
            pltpu.VMEM((_CH,), jnp.float32),
            pltpu.VMEM((_HW,), jnp.float32),
            pltpu.VMEM((_HW,), jnp.float32),
            pltpu.VMEM((2, _NB + 16), jnp.float32),
            pltpu.VMEM((2, 2, _NB + 16), jnp.float32),
            pltpu.VMEM((16,), jnp.float32),
            pltpu.VMEM((16,), jnp.float32),
            pltpu.VMEM_SHARED((16, 2, _NB + 16), jnp.float32),
            pltpu.SemaphoreType.DMA,
            pltpu.SemaphoreType.DMA,
        ],
    )
    return select(loss_flat, kf_arr)


@jax.jit
def _topk_mean(probs, gt, kf_arr):
    row_sums = _run_pipeline(probs, gt, kf_arr)
    return jnp.sum(row_sums.reshape(_ROWS, 16)[:, 0]) / np.float32(_ROWS * _N)


def kernel(probs, gt, img_size):
    k = (img_size[0].astype(jnp.int32) * img_size[1].astype(jnp.int32) * 90) // 100
    kf_arr = jnp.full((16,), k.astype(jnp.float32), dtype=jnp.float32)
    return _topk_mean(probs, gt, kf_arr)


# speedup vs baseline: 28.3400x; 1.5124x over previous
import functools

import jax
import jax.numpy as jnp
import numpy as np
from jax import lax
from jax.experimental import pallas as pl
from jax.experimental.pallas import tpu as pltpu
from jax.experimental.pallas import tpu_sc as plsc

_H = 2048
_W = 128
_N = _H * _W
_ROWS = 16
_HALF = _N // 2
_CH = 16384
_NCHUNK = _HALF // _CH
_NB = 2048
_LS = _NB + 1
_HW = 513 * 64

_LOG_EPS = np.float32(np.log(1e-7))
_LOG_1MEPS = np.float32(np.log(1.0 - 1e-7))
_I0 = np.int32(0)


def _loop32(n, body, init):
    def sb(carry, _):
        j, st = carry
        return (j + np.int32(1), body(j, st)), None

    (_, out), _ = lax.scan(sb, (jnp.int32(0), init), None, length=n)
    return out


def _loss_kernel(p0_ref, p1_ref, gt_ref, out_ref):
    d = p1_ref[0] - p0_ref[0]
    sp_tail = jnp.log1p(jnp.exp(-jnp.abs(d)))
    sp_d = jnp.maximum(d, 0.0) + sp_tail
    sp_nd = sp_d - d
    log_pc = jnp.clip(-sp_nd, _LOG_EPS, _LOG_1MEPS)
    log_1mpc = jnp.clip(-sp_d, _LOG_EPS, _LOG_1MEPS)
    g = gt_ref[0]
    out_ref[0] = -(g * log_pc) - (1.0 - g) * log_1mpc


def _select_body(loss_hbm, kf_hbm, out_hbm, buf0, buf1, hist_cnt, hist_sum,
                 red, comb, kf_v, outv, shared, sem0, sem1):
    s = lax.axis_index("s")
    c = lax.axis_index("c")
    i32 = np.int32
    half = s % i32(2)
    lr = s // i32(2)
    r = c * i32(8) + lr
    base = r * i32(_N) + half * i32(_HALF)

    iota = lax.iota(jnp.int32, 16)
    zeros16 = jnp.zeros((16,), jnp.float32)
    ones16 = jnp.ones((16,), jnp.float32)
    izeros16 = jnp.zeros((16,), jnp.int32)

    pltpu.sync_copy(kf_hbm, kf_v)
    kf = kf_v[...]

    def zero_hists(both):
        def zb(j, carry):
            for u in range(4):
                off = j * i32(64) + i32(u * 16)
                hist_cnt[pl.ds(off, 16)] = zeros16
                if both:
                    hist_sum[pl.ds(off, 16)] = zeros16
            return carry

        _loop32(_HW // 64, zb, jnp.int32(0))

    def scatter_pass(bstar_vec):
        bufs = (buf0, buf1)
        sems = (sem0, sem1)
        handles = {}
        handles[0] = pltpu.async_copy(
            loss_hbm.at[pl.ds(base, _CH)], bufs[0], sems[0])
        acc = (zeros16, zeros16)
        for chunk in range(_NCHUNK):
            cur = chunk % 2
            if chunk + 1 < _NCHUNK:
                handles[chunk + 1] = pltpu.async_copy(
                    loss_hbm.at[pl.ds(base + i32((chunk + 1) * _CH), _CH)],
                    bufs[(chunk + 1) % 2], sems[(chunk + 1) % 2])
            handles[chunk].wait()
            bref = bufs[cur]

            def body(j, carry):
                ac, asum = carry
                vs = [bref[pl.ds(j * i32(128) + i32(u * 16), 16)]
                      for u in range(8)]
                bits = [plsc.bitcast(v, jnp.int32) for v in vs]
                tops = [b >> i32(20) for b in bits]
                if bstar_vec is None:
                    idxs = [iota * i32(_LS) + t for t in tops]
                    for idx in idxs:
                        plsc.addupdate_scatter(hist_cnt, [idx], ones16)
                else:
                    ms = [t == bstar_vec for t in tops]
                    gms = [t > bstar_vec for t in tops]
                    idxs = [iota * i32(_LS) + ((b >> i32(9)) & i32(_NB - 1))
                            for b in bits]
                    for u in range(8):
                        plsc.addupdate_scatter(hist_cnt, [idxs[u]], ones16,
                                               mask=ms[u])
                        plsc.addupdate_scatter(hist_sum, [idxs[u]], vs[u],
                                               mask=ms[u])
                    for u in range(8):
                        ac = ac + jnp.where(gms[u], ones16, zeros16)
                        asum = asum + jnp.where(gms[u], vs[u], zeros16)
                return (ac, asum)

            acc = _loop32(_CH // 128, body, acc)
        return acc

    def lane_reduce_and_stage(acc, do_sum):
        def body(j, carry):
            acc_c = zeros16
            acc_s = zeros16
            for l in range(16):
                off = i32(l * _LS) + j * i32(16)
                acc_c = acc_c + hist_cnt[pl.ds(off, 16)]
                if do_sum:
                    acc_s = acc_s + hist_sum[pl.ds(off, 16)]
                else:
                    hist_cnt[pl.ds(off, 16)] = zeros16
            red[0, pl.ds(j * i32(16), 16)] = acc_c
            if do_sum:
                red[1, pl.ds(j * i32(16), 16)] = acc_s
            return carry

        _loop32(_NB // 16, body, jnp.int32(0))
        red[0, pl.ds(_NB, 16)] = acc[0]
        red[1, pl.ds(_NB, 16)] = acc[1]
        pltpu.sync_copy(red, shared.at[s])

    def scan_desc(kf_vec, with_sums):
        def body(i, st):
            found, bbase, selc, sels, selS, selSS, selpc, carc, cars = st
            j = i32(_NB // 16 - 1) - i
            c_cnt = (comb[0, 0, pl.ds(j * i32(16), 16)]
                     + comb[1, 0, pl.ds(j * i32(16), 16)])
            sfx_c = jnp.flip(plsc.cumsum(jnp.flip(c_cnt))) + carc
            mask = sfx_c >= kf_vec
            pc = plsc.all_reduce_population_count(mask)
            hit = (pc > i32(0)) & (found == i32(0))
            found = jnp.where(hit, jnp.int32(1), found)
            bbase = jnp.where(hit, jnp.full((16,), j, jnp.int32), bbase)
            selc = jnp.where(hit, c_cnt, selc)
            selS = jnp.where(hit, sfx_c, selS)
            selpc = jnp.where(hit, pc, selpc)
            carc = carc + jnp.full((16,), jnp.sum(c_cnt), jnp.float32)
            if with_sums:
                c_sum = (comb[0, 1, pl.ds(j * i32(16), 16)]
                         + comb[1, 1, pl.ds(j * i32(16), 16)])
                sfx_s = jnp.flip(plsc.cumsum(jnp.flip(c_sum))) + cars
                sels = jnp.where(hit, c_sum, sels)
                selSS = jnp.where(hit, sfx_s, selSS)
                cars = cars + jnp.full((16,), jnp.sum(c_sum), jnp.float32)
            return (found, bbase, selc, sels, selS, selSS, selpc, carc, cars)

        st0 = (izeros16, izeros16, zeros16, zeros16, zeros16, zeros16,
               izeros16, zeros16, zeros16)
        (_, bbase, selc, sels, selS, selSS, selpc, _, _) = _loop32(
            _NB // 16, body, st0)
        lane = selpc - i32(1)
        lm = iota == lane
        cntb = jnp.full((16,), jnp.sum(jnp.where(lm, selc, zeros16)),
                        jnp.float32)
        sumb = jnp.full((16,), jnp.sum(jnp.where(lm, sels, zeros16)),
                        jnp.float32)
        s_at = jnp.full((16,), jnp.sum(jnp.where(lm, selS, zeros16)),
                        jnp.float32)
        ss_at = jnp.full((16,), jnp.sum(jnp.where(lm, selSS, zeros16)),
                         jnp.float32)
        bstar = bbase * i32(16) + lane
        return bstar, s_at - cntb, ss_at - sumb, cntb, sumb

    pair = s - half

    zero_hists(both=True)
    scatter_pass(None)
    lane_reduce_and_stage((zeros16, zeros16), do_sum=False)
    plsc.subcore_barrier()

    pltpu.sync_copy(shared.at[pl.ds(pair, 2)], comb)
    bstar, _, _, _, _ = scan_desc(kf, with_sums=False)
    plsc.subcore_barrier()

    acc = scatter_pass(bstar)
    lane_reduce_and_stage(acc, do_sum=True)
    plsc.subcore_barrier()

    pltpu.sync_copy(shared.at[pl.ds(pair, 2)], comb)
    tail_c = comb[0, 0, pl.ds(_NB, 16)] + comb[1, 0, pl.ds(_NB, 16)]
    tail_s = comb[0, 1, pl.ds(_NB, 16)] + comb[1, 1, pl.ds(_NB, 16)]
    cnt_gt = jnp.full((16,), jnp.sum(tail_c), jnp.float32)
    sum_gt = jnp.full((16,), jnp.sum(tail_s), jnp.float32)
    kf2 = kf - cnt_gt
    sb_star, cnt2_gt, sum2_gt, cnt2b, sum2b = scan_desc(kf2, with_sums=True)
    remainder = kf2 - cnt2_gt
    avg = sum2b / jnp.maximum(cnt2b, 1.0)
    row = sum_gt + sum2_gt + remainder * avg
    row = jnp.where(kf <= 0.0, zeros16, row)

    @pl.when(half == 0)
    def _():
        outv[...] = row
        pltpu.sync_copy(outv, out_hbm.at[pl.ds(r * i32(16), 16)])


@jax.jit
def _run_pipeline(probs, gt, kf_arr):
    p0 = probs[:, 0].reshape(_ROWS, _H, _W)
    p1 = probs[:, 1].reshape(_ROWS, _H, _W)
    gtr = gt.reshape(_ROWS, _H, _W)

    loss = pl.pallas_call(
        _loss_kernel,
        grid=(_ROWS,),
        in_specs=[
            pl.BlockSpec((1, _H, _W), lambda i: (i, _I0, _I0)),
            pl.BlockSpec((1, _H, _W), lambda i: (i, _I0, _I0)),
            pl.BlockSpec((1, _H, _W), lambda i: (i, _I0, _I0)),
        ],
        out_specs=pl.BlockSpec((1, _H, _W), lambda i: (i, _I0, _I0)),
        out_shape=jax.ShapeDtypeStruct((_ROWS, _H, _W), jnp.float32),
        compiler_params=pltpu.CompilerParams(
            dimension_semantics=("arbitrary",),
        ),
    )(p0, p1, gtr)
    loss_flat = loss.reshape(_ROWS * _N)

    mesh = plsc.VectorSubcoreMesh(core_axis_name="c", subcore_axis_name="s")
    select = pl.kernel(
        _select_body,
        mesh=mesh,
        compiler_params=pltpu.CompilerParams(needs_layout_passes=False),
        out_type=jax.ShapeDtypeStruct((_ROWS * 16,), jnp.float32),
        scratch_types=[
            pltpu.VMEM((_CH,), jnp.float32),
            pltpu.VMEM((_CH,), jnp.float32),
            pltpu.VMEM((_HW,), jnp.float32),
            pltpu.VMEM((_HW,), jnp.float32),
            pltpu.VMEM((2, _NB + 16), jnp.float32),
            pltpu.VMEM((2, 2, _NB + 16), jnp.float32),
            pltpu.VMEM((16,), jnp.float32),
            pltpu.VMEM((16,), jnp.float32),
            pltpu.VMEM_SHARED((16, 2, _NB + 16), jnp.float32),
            pltpu.SemaphoreType.DMA,
            pltpu.SemaphoreType.DMA,
        ],
    )
    return select(loss_flat, kf_arr)


@jax.jit
def _topk_mean(probs, gt, kf_arr):
    row_sums = _run_pipeline(probs, gt, kf_arr)
    return jnp.sum(row_sums.reshape(_ROWS, 16)[:, 0]) / np.float32(_ROWS * _N)


def kernel(probs, gt, img_size):
    k = (img_size[0].astype(jnp.int32) * img_size[1].astype(jnp.int32) * 90) // 100
    kf_arr = jnp.full((16,), k.astype(jnp.float32), dtype=jnp.float32)
    return _topk_mean(probs, gt, kf_arr)


# speedup vs baseline: 32.7254x; 1.1547x over previous
import functools

import jax
import jax.numpy as jnp
import numpy as np
from jax import lax
from jax.experimental import pallas as pl
from jax.experimental.pallas import tpu as pltpu
from jax.experimental.pallas import tpu_sc as plsc

_H = 2048
_W = 128
_N = _H * _W
_ROWS = 16
_HALF = _N // 2
_CH = 16384
_NCHUNK = _HALF // _CH
_NB = 2048
_LS = _NB + 1
_HW = 513 * 64

_LOG_EPS = np.float32(np.log(1e-7))
_LOG_1MEPS = np.float32(np.log(1.0 - 1e-7))
_I0 = np.int32(0)


def _loop32(n, body, init):
    def sb(carry, _):
        j, st = carry
        return (j + np.int32(1), body(j, st)), None

    (_, out), _ = lax.scan(sb, (jnp.int32(0), init), None, length=n)
    return out


def _loss_kernel(p_ref, gt_ref, out_ref):
    d = p_ref[0, 1] - p_ref[0, 0]
    sp_tail = jnp.log1p(jnp.exp(-jnp.abs(d)))
    sp_d = jnp.maximum(d, 0.0) + sp_tail
    sp_nd = sp_d - d
    log_pc = jnp.clip(-sp_nd, _LOG_EPS, _LOG_1MEPS)
    log_1mpc = jnp.clip(-sp_d, _LOG_EPS, _LOG_1MEPS)
    g = gt_ref[0]
    out_ref[0] = -(g * log_pc) - (1.0 - g) * log_1mpc


def _select_body(loss_hbm, kf_hbm, out_hbm, buf0, buf1, hist_cnt, hist_sum,
                 red, comb, kf_v, outv, shared, sem0, sem1):
    s = lax.axis_index("s")
    c = lax.axis_index("c")
    i32 = np.int32
    half = s % i32(2)
    lr = s // i32(2)
    r = c * i32(8) + lr
    base = r * i32(_N) + half * i32(_HALF)

    iota = lax.iota(jnp.int32, 16)
    zeros16 = jnp.zeros((16,), jnp.float32)
    ones16 = jnp.ones((16,), jnp.float32)
    izeros16 = jnp.zeros((16,), jnp.int32)

    pltpu.sync_copy(kf_hbm, kf_v)
    kf = kf_v[...]

    def zero_hists(both):
        def zb(j, carry):
            for u in range(4):
                off = j * i32(64) + i32(u * 16)
                hist_cnt[pl.ds(off, 16)] = zeros16
                if both:
                    hist_sum[pl.ds(off, 16)] = zeros16
            return carry

        _loop32(_HW // 64, zb, jnp.int32(0))

    def scatter_pass(bstar_vec):
        bufs = (buf0, buf1)
        sems = (sem0, sem1)
        handles = {}
        handles[0] = pltpu.async_copy(
            loss_hbm.at[pl.ds(base, _CH)], bufs[0], sems[0])
        acc = (zeros16, zeros16)
        for chunk in range(_NCHUNK):
            cur = chunk % 2
            if chunk + 1 < _NCHUNK:
                handles[chunk + 1] = pltpu.async_copy(
                    loss_hbm.at[pl.ds(base + i32((chunk + 1) * _CH), _CH)],
                    bufs[(chunk + 1) % 2], sems[(chunk + 1) % 2])
            handles[chunk].wait()
            bref = bufs[cur]

            def body(j, carry):
                ac, asum = carry
                vs = [bref[pl.ds(j * i32(128) + i32(u * 16), 16)]
                      for u in range(8)]
                bits = [plsc.bitcast(v, jnp.int32) for v in vs]
                tops = [b >> i32(20) for b in bits]
                if bstar_vec is None:
                    idxs = [iota * i32(_LS) + t for t in tops]
                    for idx in idxs:
                        plsc.addupdate_scatter(hist_cnt, [idx], ones16)
                else:
                    ms = [t == bstar_vec for t in tops]
                    gms = [t > bstar_vec for t in tops]
                    idxs = [iota * i32(_LS) + ((b >> i32(9)) & i32(_NB - 1))
                            for b in bits]
                    for u in range(8):
                        plsc.addupdate_scatter(hist_cnt, [idxs[u]], ones16,
                                               mask=ms[u])
                        plsc.addupdate_scatter(hist_sum, [idxs[u]], vs[u],
                                               mask=ms[u])
                    for u in range(8):
                        ac = ac + jnp.where(gms[u], ones16, zeros16)
                        asum = asum + jnp.where(gms[u], vs[u], zeros16)
                return (ac, asum)

            acc = _loop32(_CH // 128, body, acc)
        return acc

    def lane_reduce_and_stage(acc, do_sum):
        def body(j, carry):
            acc_c = zeros16
            acc_s = zeros16
            for l in range(16):
                off = i32(l * _LS) + j * i32(16)
                acc_c = acc_c + hist_cnt[pl.ds(off, 16)]
                if do_sum:
                    acc_s = acc_s + hist_sum[pl.ds(off, 16)]
                else:
                    hist_cnt[pl.ds(off, 16)] = zeros16
            red[0, pl.ds(j * i32(16), 16)] = acc_c
            if do_sum:
                red[1, pl.ds(j * i32(16), 16)] = acc_s
            return carry

        _loop32(_NB // 16, body, jnp.int32(0))
        red[0, pl.ds(_NB, 16)] = acc[0]
        red[1, pl.ds(_NB, 16)] = acc[1]
        pltpu.sync_copy(red, shared.at[s])

    def scan_desc(kf_vec, with_sums):
        def body(i, st):
            found, bbase, selc, sels, selS, selSS, selpc, carc, cars = st
            j = i32(_NB // 16 - 1) - i
            c_cnt = (comb[0, 0, pl.ds(j * i32(16), 16)]
                     + comb[1, 0, pl.ds(j * i32(16), 16)])
            sfx_c = jnp.flip(plsc.cumsum(jnp.flip(c_cnt))) + carc
            mask = sfx_c >= kf_vec
            pc = plsc.all_reduce_population_count(mask)
            hit = (pc > i32(0)) & (found == i32(0))
            found = jnp.where(hit, jnp.int32(1), found)
            bbase = jnp.where(hit, jnp.full((16,), j, jnp.int32), bbase)
            selc = jnp.where(hit, c_cnt, selc)
            selS = jnp.where(hit, sfx_c, selS)
            selpc = jnp.where(hit, pc, selpc)
            carc = carc + jnp.full((16,), jnp.sum(c_cnt), jnp.float32)
            if with_sums:
                c_sum = (comb[0, 1, pl.ds(j * i32(16), 16)]
                         + comb[1, 1, pl.ds(j * i32(16), 16)])
                sfx_s = jnp.flip(plsc.cumsum(jnp.flip(c_sum))) + cars
                sels = jnp.where(hit, c_sum, sels)
                selSS = jnp.where(hit, sfx_s, selSS)
                cars = cars + jnp.full((16,), jnp.sum(c_sum), jnp.float32)
            return (found, bbase, selc, sels, selS, selSS, selpc, carc, cars)

        st0 = (izeros16, izeros16, zeros16, zeros16, zeros16, zeros16,
               izeros16, zeros16, zeros16)
        (_, bbase, selc, sels, selS, selSS, selpc, _, _) = _loop32(
            _NB // 16, body, st0)
        lane = selpc - i32(1)
        lm = iota == lane
        cntb = jnp.full((16,), jnp.sum(jnp.where(lm, selc, zeros16)),
                        jnp.float32)
        sumb = jnp.full((16,), jnp.sum(jnp.where(lm, sels, zeros16)),
                        jnp.float32)
        s_at = jnp.full((16,), jnp.sum(jnp.where(lm, selS, zeros16)),
                        jnp.float32)
        ss_at = jnp.full((16,), jnp.sum(jnp.where(lm, selSS, zeros16)),
                         jnp.float32)
        bstar = bbase * i32(16) + lane
        return bstar, s_at - cntb, ss_at - sumb, cntb, sumb

    pair = s - half

    zero_hists(both=True)
    scatter_pass(None)
    lane_reduce_and_stage((zeros16, zeros16), do_sum=False)
    plsc.subcore_barrier()

    pltpu.sync_copy(shared.at[pl.ds(pair, 2)], comb)
    bstar, _, _, _, _ = scan_desc(kf, with_sums=False)
    plsc.subcore_barrier()

    acc = scatter_pass(bstar)
    lane_reduce_and_stage(acc, do_sum=True)
    plsc.subcore_barrier()

    pltpu.sync_copy(shared.at[pl.ds(pair, 2)], comb)
    tail_c = comb[0, 0, pl.ds(_NB, 16)] + comb[1, 0, pl.ds(_NB, 16)]
    tail_s = comb[0, 1, pl.ds(_NB, 16)] + comb[1, 1, pl.ds(_NB, 16)]
    cnt_gt = jnp.full((16,), jnp.sum(tail_c), jnp.float32)
    sum_gt = jnp.full((16,), jnp.sum(tail_s), jnp.float32)
    kf2 = kf - cnt_gt
    sb_star, cnt2_gt, sum2_gt, cnt2b, sum2b = scan_desc(kf2, with_sums=True)
    remainder = kf2 - cnt2_gt
    avg = sum2b / jnp.maximum(cnt2b, 1.0)
    row = sum_gt + sum2_gt + remainder * avg
    row = jnp.where(kf <= 0.0, zeros16, row)

    @pl.when(half == 0)
    def _():
        outv[...] = row
        pltpu.sync_copy(outv, out_hbm.at[pl.ds(r * i32(16), 16)])


@jax.jit
def _run_pipeline(probs, gt, kf_arr):
    pr = probs.reshape(_ROWS, 2, _H, _W)
    gtr = gt.reshape(_ROWS, _H, _W)

    loss = pl.pallas_call(
        _loss_kernel,
        grid=(_ROWS,),
        in_specs=[
            pl.BlockSpec((1, 2, _H, _W), lambda i: (i, _I0, _I0, _I0)),
            pl.BlockSpec((1, _H, _W), lambda i: (i, _I0, _I0)),
        ],
        out_specs=pl.BlockSpec((1, _H, _W), lambda i: (i, _I0, _I0)),
        out_shape=jax.ShapeDtypeStruct((_ROWS, _H, _W), jnp.float32),
        compiler_params=pltpu.CompilerParams(
            dimension_semantics=("arbitrary",),
        ),
    )(pr, gtr)
    loss_flat = loss.reshape(_ROWS * _N)

    mesh = plsc.VectorSubcoreMesh(core_axis_name="c", subcore_axis_name="s")
    select = pl.kernel(
        _select_body,
        mesh=mesh,
        compiler_params=pltpu.CompilerParams(needs_layout_passes=False),
        out_type=jax.ShapeDtypeStruct((_ROWS * 16,), jnp.float32),
        scratch_types=[
            pltpu.VMEM((_CH,), jnp.float32),
            pltpu.VMEM((_CH,), jnp.float32),
            pltpu.VMEM((_HW,), jnp.float32),
            pltpu.VMEM((_HW,), jnp.float32),
            pltpu.VMEM((2, _NB + 16), jnp.float32),
            pltpu.VMEM((2, 2, _NB + 16), jnp.float32),
            pltpu.VMEM((16,), jnp.float32),
            pltpu.VMEM((16,), jnp.float32),
            pltpu.VMEM_SHARED((16, 2, _NB + 16), jnp.float32),
            pltpu.SemaphoreType.DMA,
            pltpu.SemaphoreType.DMA,
        ],
    )
    return select(loss_flat, kf_arr)


@jax.jit
def _topk_mean(probs, gt, kf_arr):
    row_sums = _run_pipeline(probs, gt, kf_arr)
    return jnp.sum(row_sums.reshape(_ROWS, 16)[:, 0]) / np.float32(_ROWS * _N)


def kernel(probs, gt, img_size):
    k = (img_size[0].astype(jnp.int32) * img_size[1].astype(jnp.int32) * 90) // 100
    kf_arr = jnp.full((16,), k.astype(jnp.float32), dtype=jnp.float32)
    return _topk_mean(probs, gt, kf_arr)


# speedup vs baseline: 33.5812x; 1.0262x over previous
import functools

import jax
import jax.numpy as jnp
import numpy as np
from jax import lax
from jax.experimental import pallas as pl
from jax.experimental.pallas import tpu as pltpu
from jax.experimental.pallas import tpu_sc as plsc

_H = 2048
_W = 128
_N = _H * _W
_ROWS = 16
_HALF = _N // 2
_CH = 16384
_NCHUNK = _HALF // _CH
_NB = 2048
_LS = _NB + 1
_HW = 513 * 64

_LOG_EPS = np.float32(np.log(1e-7))
_LOG_1MEPS = np.float32(np.log(1.0 - 1e-7))
_I0 = np.int32(0)


def _loop32(n, body, init):
    def sb(carry, _):
        j, st = carry
        return (j + np.int32(1), body(j, st)), None

    (_, out), _ = lax.scan(sb, (jnp.int32(0), init), None, length=n)
    return out


def _loss_kernel(p_ref, gt_ref, out_ref):
    d = p_ref[:, 1] - p_ref[:, 0]
    sp_tail = jnp.log1p(jnp.exp(-jnp.abs(d)))
    sp_d = jnp.maximum(d, 0.0) + sp_tail
    sp_nd = sp_d - d
    log_pc = jnp.clip(-sp_nd, _LOG_EPS, _LOG_1MEPS)
    log_1mpc = jnp.clip(-sp_d, _LOG_EPS, _LOG_1MEPS)
    g = gt_ref[...]
    out_ref[...] = -(g * log_pc) - (1.0 - g) * log_1mpc


def _select_body(loss_hbm, kf_hbm, out_hbm, buf0, buf1, hist_cnt, hist_sum,
                 red, comb, kf_v, outv, shared, sem0, sem1):
    s = lax.axis_index("s")
    c = lax.axis_index("c")
    i32 = np.int32
    half = s % i32(2)
    lr = s // i32(2)
    r = c * i32(8) + lr
    base = r * i32(_N) + half * i32(_HALF)

    iota = lax.iota(jnp.int32, 16)
    zeros16 = jnp.zeros((16,), jnp.float32)
    ones16 = jnp.ones((16,), jnp.float32)
    izeros16 = jnp.zeros((16,), jnp.int32)

    pltpu.sync_copy(kf_hbm, kf_v)
    kf = kf_v[...]

    def zero_hists(both):
        def zb(j, carry):
            for u in range(4):
                off = j * i32(64) + i32(u * 16)
                hist_cnt[pl.ds(off, 16)] = zeros16
                if both:
                    hist_sum[pl.ds(off, 16)] = zeros16
            return carry

        _loop32(_HW // 64, zb, jnp.int32(0))

    def scatter_pass(bstar_vec):
        bufs = (buf0, buf1)
        sems = (sem0, sem1)
        handles = {}
        handles[0] = pltpu.async_copy(
            loss_hbm.at[pl.ds(base, _CH)], bufs[0], sems[0])
        acc = (zeros16, zeros16)
        for chunk in range(_NCHUNK):
            cur = chunk % 2
            if chunk + 1 < _NCHUNK:
                handles[chunk + 1] = pltpu.async_copy(
                    loss_hbm.at[pl.ds(base + i32((chunk + 1) * _CH), _CH)],
                    bufs[(chunk + 1) % 2], sems[(chunk + 1) % 2])
            handles[chunk].wait()
            bref = bufs[cur]

            def body(j, carry):
                ac, asum = carry
                vs = [bref[pl.ds(j * i32(128) + i32(u * 16), 16)]
                      for u in range(8)]
                bits = [plsc.bitcast(v, jnp.int32) for v in vs]
                tops = [b >> i32(20) for b in bits]
                if bstar_vec is None:
                    idxs = [iota * i32(_LS) + t for t in tops]
                    for idx in idxs:
                        plsc.addupdate_scatter(hist_cnt, [idx], ones16)
                else:
                    ms = [t == bstar_vec for t in tops]
                    gms = [t > bstar_vec for t in tops]
                    idxs = [iota * i32(_LS) + ((b >> i32(9)) & i32(_NB - 1))
                            for b in bits]
                    for u in range(8):
                        plsc.addupdate_scatter(hist_cnt, [idxs[u]], ones16,
                                               mask=ms[u])
                        plsc.addupdate_scatter(hist_sum, [idxs[u]], vs[u],
                                               mask=ms[u])
                    for u in range(8):
                        ac = ac + jnp.where(gms[u], ones16, zeros16)
                        asum = asum + jnp.where(gms[u], vs[u], zeros16)
                return (ac, asum)

            acc = _loop32(_CH // 128, body, acc)
        return acc

    def lane_reduce_and_stage(acc, do_sum):
        def body(j, carry):
            acc_c = zeros16
            acc_s = zeros16
            for l in range(16):
                off = i32(l * _LS) + j * i32(16)
                acc_c = acc_c + hist_cnt[pl.ds(off, 16)]
                if do_sum:
                    acc_s = acc_s + hist_sum[pl.ds(off, 16)]
                else:
                    hist_cnt[pl.ds(off, 16)] = zeros16
            red[0, pl.ds(j * i32(16), 16)] = acc_c
            if do_sum:
                red[1, pl.ds(j * i32(16), 16)] = acc_s
            return carry

        _loop32(_NB // 16, body, jnp.int32(0))
        red[0, pl.ds(_NB, 16)] = acc[0]
        red[1, pl.ds(_NB, 16)] = acc[1]
        pltpu.sync_copy(red, shared.at[s])

    def scan_desc(kf_vec, with_sums):
        def body(i, st):
            found, bbase, selc, sels, selS, selSS, selpc, carc, cars = st
            j = i32(_NB // 16 - 1) - i
            c_cnt = (comb[0, 0, pl.ds(j * i32(16), 16)]
                     + comb[1, 0, pl.ds(j * i32(16), 16)])
            sfx_c = jnp.flip(plsc.cumsum(jnp.flip(c_cnt))) + carc
            mask = sfx_c >= kf_vec
            pc = plsc.all_reduce_population_count(mask)
            hit = (pc > i32(0)) & (found == i32(0))
            found = jnp.where(hit, jnp.int32(1), found)
            bbase = jnp.where(hit, jnp.full((16,), j, jnp.int32), bbase)
            selc = jnp.where(hit, c_cnt, selc)
            selS = jnp.where(hit, sfx_c, selS)
            selpc = jnp.where(hit, pc, selpc)
            carc = carc + jnp.full((16,), jnp.sum(c_cnt), jnp.float32)
            if with_sums:
                c_sum = (comb[0, 1, pl.ds(j * i32(16), 16)]
                         + comb[1, 1, pl.ds(j * i32(16), 16)])
                sfx_s = jnp.flip(plsc.cumsum(jnp.flip(c_sum))) + cars
                sels = jnp.where(hit, c_sum, sels)
                selSS = jnp.where(hit, sfx_s, selSS)
                cars = cars + jnp.full((16,), jnp.sum(c_sum), jnp.float32)
            return (found, bbase, selc, sels, selS, selSS, selpc, carc, cars)

        st0 = (izeros16, izeros16, zeros16, zeros16, zeros16, zeros16,
               izeros16, zeros16, zeros16)
        (_, bbase, selc, sels, selS, selSS, selpc, _, _) = _loop32(
            _NB // 16, body, st0)
        lane = selpc - i32(1)
        lm = iota == lane
        cntb = jnp.full((16,), jnp.sum(jnp.where(lm, selc, zeros16)),
                        jnp.float32)
        sumb = jnp.full((16,), jnp.sum(jnp.where(lm, sels, zeros16)),
                        jnp.float32)
        s_at = jnp.full((16,), jnp.sum(jnp.where(lm, selS, zeros16)),
                        jnp.float32)
        ss_at = jnp.full((16,), jnp.sum(jnp.where(lm, selSS, zeros16)),
                         jnp.float32)
        bstar = bbase * i32(16) + lane
        return bstar, s_at - cntb, ss_at - sumb, cntb, sumb

    pair = s - half

    zero_hists(both=True)
    scatter_pass(None)
    lane_reduce_and_stage((zeros16, zeros16), do_sum=False)
    plsc.subcore_barrier()

    pltpu.sync_copy(shared.at[pl.ds(pair, 2)], comb)
    bstar, _, _, _, _ = scan_desc(kf, with_sums=False)
    plsc.subcore_barrier()

    acc = scatter_pass(bstar)
    lane_reduce_and_stage(acc, do_sum=True)
    plsc.subcore_barrier()

    pltpu.sync_copy(shared.at[pl.ds(pair, 2)], comb)
    tail_c = comb[0, 0, pl.ds(_NB, 16)] + comb[1, 0, pl.ds(_NB, 16)]
    tail_s = comb[0, 1, pl.ds(_NB, 16)] + comb[1, 1, pl.ds(_NB, 16)]
    cnt_gt = jnp.full((16,), jnp.sum(tail_c), jnp.float32)
    sum_gt = jnp.full((16,), jnp.sum(tail_s), jnp.float32)
    kf2 = kf - cnt_gt
    sb_star, cnt2_gt, sum2_gt, cnt2b, sum2b = scan_desc(kf2, with_sums=True)
    remainder = kf2 - cnt2_gt
    avg = sum2b / jnp.maximum(cnt2b, 1.0)
    row = sum_gt + sum2_gt + remainder * avg
    row = jnp.where(kf <= 0.0, zeros16, row)

    @pl.when(half == 0)
    def _():
        outv[...] = row
        pltpu.sync_copy(outv, out_hbm.at[pl.ds(r * i32(16), 16)])


@jax.jit
def _run_pipeline(probs, gt, kf_arr):
    pr = probs.reshape(_ROWS, 2, _H, _W)
    gtr = gt.reshape(_ROWS, _H, _W)

    rpb = 4
    loss = pl.pallas_call(
        _loss_kernel,
        grid=(_ROWS // rpb,),
        in_specs=[
            pl.BlockSpec((rpb, 2, _H, _W), lambda i: (i, _I0, _I0, _I0)),
            pl.BlockSpec((rpb, _H, _W), lambda i: (i, _I0, _I0)),
        ],
        out_specs=pl.BlockSpec((rpb, _H, _W), lambda i: (i, _I0, _I0)),
        out_shape=jax.ShapeDtypeStruct((_ROWS, _H, _W), jnp.float32),
        compiler_params=pltpu.CompilerParams(
            dimension_semantics=("parallel",),
        ),
    )(pr, gtr)
    loss_flat = loss.reshape(_ROWS * _N)

    mesh = plsc.VectorSubcoreMesh(core_axis_name="c", subcore_axis_name="s")
    select = pl.kernel(
        _select_body,
        mesh=mesh,
        compiler_params=pltpu.CompilerParams(needs_layout_passes=False),
        out_type=jax.ShapeDtypeStruct((_ROWS * 16,), jnp.float32),
        scratch_types=[
            pltpu.VMEM((_CH,), jnp.float32),
            pltpu.VMEM((_CH,), jnp.float32),
            pltpu.VMEM((_HW,), jnp.float32),
            pltpu.VMEM((_HW,), jnp.float32),
            pltpu.VMEM((2, _NB + 16), jnp.float32),
            pltpu.VMEM((2, 2, _NB + 16), jnp.float32),
            pltpu.VMEM((16,), jnp.float32),
            pltpu.VMEM((16,), jnp.float32),
            pltpu.VMEM_SHARED((16, 2, _NB + 16), jnp.float32),
            pltpu.SemaphoreType.DMA,
            pltpu.SemaphoreType.DMA,
        ],
    )
    return select(loss_flat, kf_arr)


@jax.jit
def _topk_mean(probs, gt, kf_arr):
    row_sums = _run_pipeline(probs, gt, kf_arr)
    return jnp.sum(row_sums.reshape(_ROWS, 16)[:, 0]) / np.float32(_ROWS * _N)


def kernel(probs, gt, img_size):
    k = (img_size[0].astype(jnp.int32) * img_size[1].astype(jnp.int32) * 90) // 100
    kf_arr = jnp.full((16,), k.astype(jnp.float32), dtype=jnp.float32)
    return _topk_mean(probs, gt, kf_arr)
